# baseline (speedup 1.0000x reference)
import functools

import jax
import jax.numpy as jnp
from jax import lax
from jax.experimental import pallas as pl
from jax.experimental.pallas import tpu as pltpu
from jax.experimental.pallas import tpu_sc as plsc

B, L, D = 16384, 50, 128
VOCAB = 24
MAX_CHARGE = 10
ROWS = L * VOCAB * MAX_CHARGE
CAT = L + VOCAB + MAX_CHARGE

NC, NS = 2, 16
NW = NC * NS
TOK = B * L
TOK_PER_W = TOK // NW
PTOK = 128
NPIECE = TOK_PER_W // PTOK


def _pos_enc():
    pos = jnp.arange(L, dtype=jnp.float32)[:, None]
    i = jnp.arange(D // 2, dtype=jnp.float32)[None, :]
    angle = pos / jnp.power(10000.0, (2.0 * i) / D)
    return jnp.stack([jnp.sin(angle), jnp.cos(angle)], axis=-1).reshape(L, D)


def _tc_prep(cat_ref, mega_ref):
    r = lax.broadcasted_iota(jnp.int32, (ROWS, CAT), 0)
    col = lax.broadcasted_iota(jnp.int32, (ROWS, CAT), 1)
    l = r // (VOCAB * MAX_CHARGE)
    v = (r // MAX_CHARGE) % VOCAB
    c = r % MAX_CHARGE
    oh = ((col == l) | (col == L + v) | (col == L + VOCAB + c)).astype(jnp.float32)
    mega_ref[...] = jnp.dot(oh, cat_ref[...], preferred_element_type=jnp.float32)


def _sc_gather(
    mega_hbm, tok_hbm, ch_hbm, out_hbm,
    mega_sp, tok0, tok1, ch0, ch1, idx0, idx1, buf0, buf1,
    l0, l1, g0, g1, s0, s1,
):
    sid = lax.axis_index("s")
    wid = sid * NC + lax.axis_index("c")
    base = wid * TOK_PER_W

    pltpu.async_copy(tok_hbm.at[pl.ds(base, PTOK)], tok0, l0)
    pltpu.async_copy(ch_hbm.at[pl.ds(base % B, PTOK)], ch0, l0)
    pltpu.async_copy(tok_hbm.at[pl.ds(base + PTOK, PTOK)], tok1, l1)
    pltpu.async_copy(ch_hbm.at[pl.ds((base + PTOK) % B, PTOK)], ch1, l1)

    @pl.when(sid < 10)
    def _():
        pltpu.sync_copy(
            mega_hbm.at[pl.ds(sid * 1200, 1200)],
            mega_sp.at[pl.ds(sid * 1200, 1200)],
        )

    plsc.subcore_barrier()

    def piece(i, p, tokb, chb, idxb, buf, lsem, gsem, ssem):
        @pl.when(i > 0)
        def _():
            pltpu.make_async_copy(buf, out_hbm.at[pl.ds(base, PTOK)], ssem).wait()

        pltpu.make_async_copy(tok_hbm.at[pl.ds(base, PTOK)], tokb, lsem).wait()
        pltpu.make_async_copy(ch_hbm.at[pl.ds(base, PTOK)], chb, lsem).wait()
        r0 = base + p * PTOK
        l240 = (r0 // B) * (VOCAB * MAX_CHARGE)
        for j in range(PTOK // 16):
            sl = pl.ds(j * 16, 16)
            idxb[sl] = tokb[sl] * MAX_CHARGE + (chb[sl] + l240)
        pltpu.async_copy(mega_sp.at[idxb], buf, gsem).wait()
        pltpu.async_copy(buf, out_hbm.at[pl.ds(r0, PTOK)], ssem)

        @pl.when(p + 2 < NPIECE)
        def _():
            r2 = base + (p + 2) * PTOK
            pltpu.async_copy(tok_hbm.at[pl.ds(r2, PTOK)], tokb, lsem)
            pltpu.async_copy(ch_hbm.at[pl.ds(r2 % B, PTOK)], chb, lsem)

    def body(i, _):
        piece(i, 2 * i, tok0, ch0, idx0, buf0, l0, g0, s0)
        piece(i, 2 * i + 1, tok1, ch1, idx1, buf1, l1, g1, s1)
        return 0

    lax.fori_loop(0, NPIECE // 2, body, 0)
    for buf, ssem in ((buf0, s0), (buf1, s1)):
        pltpu.make_async_copy(buf, out_hbm.at[pl.ds(base, PTOK)], ssem).wait()


def kernel(tokens, charges, aa_table, charge_table):
    cat = jnp.concatenate([_pos_enc(), aa_table, charge_table], axis=0)
    tokT = tokens.T.reshape(TOK)
    mega = pl.pallas_call(
        _tc_prep,
        out_shape=jax.ShapeDtypeStruct((ROWS, D), jnp.float32),
    )(cat)

    sc = functools.partial(
        pl.kernel,
        out_type=jax.ShapeDtypeStruct((TOK, D), jnp.float32),
        mesh=plsc.VectorSubcoreMesh(
            core_axis_name="c", subcore_axis_name="s", num_cores=NC, num_subcores=NS
        ),
        scratch_types=[
            pltpu.VMEM_SHARED((ROWS, D), jnp.float32),
            pltpu.VMEM((PTOK,), jnp.int32),
            pltpu.VMEM((PTOK,), jnp.int32),
            pltpu.VMEM((PTOK,), jnp.int32),
            pltpu.VMEM((PTOK,), jnp.int32),
            pltpu.VMEM((PTOK,), jnp.int32),
            pltpu.VMEM((PTOK,), jnp.int32),
            pltpu.VMEM((PTOK, D), jnp.float32),
            pltpu.VMEM((PTOK, D), jnp.float32),
            pltpu.SemaphoreType.DMA,
            pltpu.SemaphoreType.DMA,
            pltpu.SemaphoreType.DMA,
            pltpu.SemaphoreType.DMA,
            pltpu.SemaphoreType.DMA,
            pltpu.SemaphoreType.DMA,
        ],
    )(_sc_gather)
    out_lmajor = sc(mega, tokT, charges)
    return out_lmajor.reshape(L, B, D).transpose(1, 0, 2)
